# baseline (speedup 1.0000x reference)
import jax
import jax.numpy as jnp
from jax import lax
from jax.experimental import pallas as pl

GRID_H, GRID_W, EMBED_DIM = 32, 32, 768
D = EMBED_DIM // 3
N = GRID_H * GRID_W
BLK = 256
NBLK = (N + 1 + BLK - 1) // BLK


def _pos_emb_kernel(row_ref, col_ref, time_ref, cls_ref, out_ref):
    b = pl.program_id(0)

    row8 = row_ref[pl.ds(pl.multiple_of(jnp.minimum(8 * b, GRID_H - 8), 8), 8)]
    rowu = jnp.broadcast_to(row8[:, None, :], (8, GRID_W, D)).reshape(BLK, D)
    colu = jnp.broadcast_to(col_ref[...][None, :, :],
                            (8, GRID_W, D)).reshape(BLK, D)
    timeu = time_ref[pl.ds(pl.multiple_of(jnp.minimum(BLK * b, N - BLK), 8),
                           BLK)]

    pr = row_ref[pl.ds(pl.multiple_of(jnp.maximum(8 * b - 8, 0), 8), 8)][7:8]
    pt = time_ref[pl.ds(pl.multiple_of(jnp.maximum(BLK * b - 8, 0), 8), 8)][7:8]
    pc = col_ref[GRID_W - 1:GRID_W]

    rowp = jnp.concatenate([pr, rowu[:BLK - 1]], axis=0)
    colp = jnp.concatenate([pc, colu[:BLK - 1]], axis=0)
    timep = jnp.concatenate([pt, timeu[:BLK - 1]], axis=0)

    v = jnp.concatenate([rowp, colp, timep], axis=-1)

    rid = lax.broadcasted_iota(jnp.int32, (BLK, EMBED_DIM), 0)
    clsv = jnp.broadcast_to(cls_ref[0], (BLK, EMBED_DIM))
    v = jnp.where((rid == 0) & (b == 0), clsv, v)

    out_ref[...] = v.reshape(BLK, 1, EMBED_DIM)


def kernel(x, row_embed, col_embed, time_embed, cls_token_pos):
    out = pl.pallas_call(
        _pos_emb_kernel,
        grid=(NBLK,),
        in_specs=[
            pl.BlockSpec((GRID_H, D), lambda b: (0, 0)),
            pl.BlockSpec((GRID_W, D), lambda b: (0, 0)),
            pl.BlockSpec((N, D), lambda b: (0, 0)),
            pl.BlockSpec((1, 1, EMBED_DIM), lambda b: (0, 0, 0)),
        ],
        out_specs=pl.BlockSpec((BLK, 1, EMBED_DIM), lambda b: (b, 0, 0)),
        out_shape=jax.ShapeDtypeStruct((N + 1, 1, EMBED_DIM), jnp.float32),
    )(row_embed, col_embed, time_embed, cls_token_pos)
    return out.reshape(1, N + 1, EMBED_DIM)
